# baseline (speedup 1.0000x reference)
import functools

import jax
import jax.numpy as jnp
from jax.experimental import pallas as pl


B, N, T, F_IN, F_OUT = 64, 325, 12, 64, 64


def _gcn_body(x_ref, adj_ref, wbd_ref, bt_ref, o_ref):
    h = jnp.dot(adj_ref[...], x_ref[0], preferred_element_type=jnp.float32)
    s = jnp.dot(h, wbd_ref[...], preferred_element_type=jnp.float32)
    o_ref[0] = jnp.maximum(s + bt_ref[...], 0.0)


@jax.jit
def kernel(x, adj, W, b):
    xf = x.reshape(B, N, T * F_IN)
    wbd = jnp.kron(jnp.eye(T, dtype=W.dtype), W)
    bt = jnp.tile(b, T).reshape(1, T * F_OUT)
    out = pl.pallas_call(
        _gcn_body,
        grid=(B,),
        in_specs=[
            pl.BlockSpec((1, N, T * F_IN), lambda i: (i, 0, 0)),
            pl.BlockSpec((N, N), lambda i: (0, 0)),
            pl.BlockSpec((T * F_IN, T * F_OUT), lambda i: (0, 0)),
            pl.BlockSpec((1, T * F_OUT), lambda i: (0, 0)),
        ],
        out_specs=pl.BlockSpec((1, N, T * F_OUT), lambda i: (i, 0, 0)),
        out_shape=jax.ShapeDtypeStruct((B, N, T * F_OUT), jnp.float32),
    )(xf, adj, wbd, bt)
    return out.reshape(B, N, T, F_OUT)


# speedup vs baseline: 1.0775x; 1.0775x over previous
import functools

import jax
import jax.numpy as jnp
from jax.experimental import pallas as pl


B, N, T, F_IN, F_OUT = 64, 325, 12, 64, 64


_G = 2


def _gcn_body(x_ref, adj_ref, wbd_ref, bt_ref, o_ref):
    h = jnp.dot(adj_ref[...], x_ref[0], preferred_element_type=jnp.float32)
    gw = _G * F_OUT
    for j in range(T // _G):
        s = jnp.dot(h[:, j * gw:(j + 1) * gw], wbd_ref[...],
                    preferred_element_type=jnp.float32)
        o_ref[0, :, j * gw:(j + 1) * gw] = jnp.maximum(s + bt_ref[...], 0.0)


@jax.jit
def kernel(x, adj, W, b):
    xf = x.reshape(B, N, T * F_IN)
    wbd = jnp.kron(jnp.eye(_G, dtype=W.dtype), W)
    bt = jnp.tile(b, _G).reshape(1, _G * F_OUT)
    out = pl.pallas_call(
        _gcn_body,
        grid=(B,),
        in_specs=[
            pl.BlockSpec((1, N, T * F_IN), lambda i: (i, 0, 0)),
            pl.BlockSpec((N, N), lambda i: (0, 0)),
            pl.BlockSpec((_G * F_IN, _G * F_OUT), lambda i: (0, 0)),
            pl.BlockSpec((1, _G * F_OUT), lambda i: (0, 0)),
        ],
        out_specs=pl.BlockSpec((1, N, T * F_OUT), lambda i: (i, 0, 0)),
        out_shape=jax.ShapeDtypeStruct((B, N, T * F_OUT), jnp.float32),
    )(xf, adj, wbd, bt)
    return out.reshape(B, N, T, F_OUT)
